# Initial kernel scaffold; baseline (speedup 1.0000x reference)
import jax
import jax.numpy as jnp
from jax.experimental import pallas as pl


def kernel(id, sector, stage, region, deal_size, revenue_multiple, growth_rate, profitability, team_experience, market_size, deal_table, sector_table, stage_table, region_table, W1, b1, g1, be1, W2, b2, g2, be2):
    raise NotImplementedError("write your pallas kernel here")


# speedup vs baseline: 1.6863x; 1.6863x over previous
import functools

import jax
import jax.numpy as jnp
from jax import lax
from jax.experimental import pallas as pl
from jax.experimental.pallas import tpu as pltpu
from jax.experimental.pallas import tpu_sc as plsc

B = 16384
EMB = 64
NW = 32
IDX_W = 128
ROWS_PER_W = B // NW
CHUNKS = ROWS_PER_W // IDX_W
OH = 80


def _sc_gather_body(idx_hbm, table_hbm, out_hbm, idx_v, rows_v, sem):
    wid = lax.axis_index("s") * 2 + lax.axis_index("c")
    pltpu.sync_copy(idx_hbm.at[pl.ds(wid * CHUNKS, CHUNKS)], idx_v)
    cps = [
        pltpu.async_copy(
            table_hbm.at[idx_v.at[j]], rows_v.at[pl.ds(j * IDX_W, IDX_W)], sem
        )
        for j in range(CHUNKS)
    ]
    for c in cps:
        c.wait()
    pltpu.sync_copy(rows_v, out_hbm.at[pl.ds(wid * ROWS_PER_W, ROWS_PER_W)])


def _make_sc_gather():
    return pl.kernel(
        _sc_gather_body,
        out_type=jax.ShapeDtypeStruct((B, EMB), jnp.float32),
        mesh=plsc.VectorSubcoreMesh(core_axis_name="c", subcore_axis_name="s"),
        scratch_types=[
            pltpu.VMEM((CHUNKS, IDX_W), jnp.int32),
            pltpu.VMEM((ROWS_PER_W, EMB), jnp.float32),
            pltpu.SemaphoreType.DMA,
        ],
    )


def _tc_body(id_emb_ref, sec_ref, stg_ref, reg_ref, num_ref, tbd_ref,
             w1a_ref, w1m_ref, w1n_ref, b1_ref, g1_ref, be1_ref,
             w2_ref, b2_ref, g2_ref, be2_ref, out_ref):
    f32 = jnp.float32
    iota = lax.broadcasted_iota(jnp.int32, (B, OH), 1)
    oh = (jnp.where(iota == sec_ref[:], 1.0, 0.0)
          + jnp.where(iota == stg_ref[:], 1.0, 0.0)
          + jnp.where(iota == reg_ref[:], 1.0, 0.0)).astype(f32)
    m = jnp.dot(tbd_ref[:], w1m_ref[:], preferred_element_type=f32)
    p1 = (jnp.dot(id_emb_ref[:], w1a_ref[:], preferred_element_type=f32)
          + jnp.dot(oh, m, preferred_element_type=f32)
          + jnp.dot(num_ref[:], w1n_ref[:], preferred_element_type=f32)
          + b1_ref[:])
    h = jnp.maximum(p1, 0.0)
    mu = jnp.mean(h, axis=0, keepdims=True)
    var = jnp.mean((h - mu) * (h - mu), axis=0, keepdims=True)
    h = (h - mu) / jnp.sqrt(var + 1e-5) * g1_ref[:] + be1_ref[:]
    p2 = jnp.dot(h, w2_ref[:], preferred_element_type=f32) + b2_ref[:]
    h2 = jnp.maximum(p2, 0.0)
    mu2 = jnp.mean(h2, axis=0, keepdims=True)
    var2 = jnp.mean((h2 - mu2) * (h2 - mu2), axis=0, keepdims=True)
    h2 = (h2 - mu2) / jnp.sqrt(var2 + 1e-5) * g2_ref[:] + be2_ref[:]
    nrm = jnp.sqrt(jnp.sum(h2 * h2, axis=-1, keepdims=True))
    out_ref[:] = h2 / jnp.maximum(nrm, 1e-12)


_tc_mlp = pl.pallas_call(
    _tc_body,
    out_shape=jax.ShapeDtypeStruct((B, EMB), jnp.float32),
)


def kernel(id, sector, stage, region, deal_size, revenue_multiple, growth_rate,
           profitability, team_experience, market_size, deal_table,
           sector_table, stage_table, region_table, W1, b1, g1, be1,
           W2, b2, g2, be2):
    id_emb = jnp.take(deal_table, id, axis=0)

    num = jnp.stack([deal_size, revenue_multiple, growth_rate, profitability,
                     team_experience, market_size], axis=-1).astype(jnp.float32)
    num = jnp.pad(num, ((0, 0), (0, 2)))
    w1n = jnp.pad(W1[112:118], ((0, 2), (0, 0)))

    tbd = jnp.zeros((OH, 48), dtype=jnp.float32)
    tbd = tbd.at[0:50, 0:16].set(sector_table)
    tbd = tbd.at[50:60, 16:32].set(stage_table)
    tbd = tbd.at[60:80, 32:48].set(region_table)

    sec = sector.astype(jnp.int32).reshape(B, 1)
    stg = stage.astype(jnp.int32).reshape(B, 1) + 50
    reg = region.astype(jnp.int32).reshape(B, 1) + 60

    return _tc_mlp(
        id_emb, sec, stg, reg, num, tbd,
        W1[0:64], W1[64:112], w1n,
        b1.reshape(1, 128), g1.reshape(1, 128), be1.reshape(1, 128),
        W2, b2.reshape(1, 64), g2.reshape(1, 64), be2.reshape(1, 64),
    )
